# Initial kernel scaffold; baseline (speedup 1.0000x reference)
import jax
import jax.numpy as jnp
from jax.experimental import pallas as pl


def kernel(x, neighbor_idx, neighbor_mask, W1, b1, W2, b2, gamma1, beta1, gamma2, beta2):
    raise NotImplementedError("write your pallas kernel here")


# speedup vs baseline: 4.2096x; 4.2096x over previous
import functools

import jax
import jax.numpy as jnp
from jax import lax
from jax.experimental import pallas as pl
from jax.experimental.pallas import tpu as pltpu
from jax.experimental.pallas import tpu_sc as plsc

N = 100000
C = 64
K = 27
KC = K // 2
EPS = 1e-4

BLK = 768
NCH = 131
NPAD = NCH * BLK
G = 128
SUB = BLK // G
NC = 2
NS = 16
NW = NC * NS


def _stats_kernel(x_ref, o_ref):
    i = pl.program_id(0)
    xb = x_ref[...]
    s = jnp.sum(xb, axis=0, keepdims=True)
    ss = jnp.sum(xb * xb, axis=0, keepdims=True)
    blk = jnp.concatenate([s, ss, jnp.zeros((6, C), jnp.float32)], axis=0)

    @pl.when(i == 0)
    def _():
        o_ref[...] = blk

    @pl.when(i != 0)
    def _():
        o_ref[...] += blk


def _stats(xp):
    return pl.pallas_call(
        _stats_kernel,
        grid=(NCH,),
        in_specs=[pl.BlockSpec((BLK, C), lambda i: (i, 0))],
        out_specs=pl.BlockSpec((8, C), lambda i: (0, 0)),
        out_shape=jax.ShapeDtypeStruct((8, C), jnp.float32),
    )(xp)


def _transform_kernel(x_ref, st_ref, gamma_ref, beta_ref, w_ref, bvec_ref, o_ref):
    i = pl.program_id(0)
    mean = st_ref[0:1, :] * (1.0 / N)
    var = st_ref[1:2, :] * (1.0 / N) - mean * mean
    rstd = lax.rsqrt(var + EPS)
    xb = x_ref[...]
    h = jnp.maximum((xb - mean) * (rstd * gamma_ref[...]) + beta_ref[...], 0.0)
    row = i * BLK + lax.broadcasted_iota(jnp.int32, (BLK, 1), 0)
    h = jnp.where(row < N, h, 0.0)
    o_ref[...] = (
        jnp.dot(h, w_ref[...], preferred_element_type=jnp.float32) + bvec_ref[...]
    )


def _transform(xp, st, gamma, beta, wr, bvec):
    return pl.pallas_call(
        _transform_kernel,
        grid=(NCH,),
        in_specs=[
            pl.BlockSpec((BLK, C), lambda i: (i, 0)),
            pl.BlockSpec((8, C), lambda i: (0, 0)),
            pl.BlockSpec((1, C), lambda i: (0, 0)),
            pl.BlockSpec((1, C), lambda i: (0, 0)),
            pl.BlockSpec((C, K * C), lambda i: (0, 0)),
            pl.BlockSpec((1, K * C), lambda i: (0, 0)),
        ],
        out_specs=pl.BlockSpec((BLK, K * C), lambda i: (i, 0)),
        out_shape=jax.ShapeDtypeStruct((NPAD, K * C), jnp.float32),
    )(xp, st, gamma.reshape(1, C), beta.reshape(1, C), wr, bvec)


def _sc_conv(hflat, idxb, init):
    mesh = plsc.VectorSubcoreMesh(core_axis_name="c", subcore_axis_name="s")

    @functools.partial(
        pl.kernel,
        out_type=jax.ShapeDtypeStruct((NPAD, C), jnp.float32),
        mesh=mesh,
        compiler_params=pltpu.CompilerParams(use_tc_tiling_on_sc=False),
        scratch_types=[
            pltpu.VMEM((K * BLK,), jnp.int32),
            pltpu.VMEM((BLK, C), jnp.float32),
            pltpu.SemaphoreType.DMA,
        ],
    )
    def conv(h_hbm, idxb_hbm, init_hbm, out_hbm, idx_v, acc_v, sem):
        cid = lax.axis_index("c")
        sid = lax.axis_index("s")
        wid = sid * NC + cid
        nch_w = 4 + jnp.where(wid < NCH - 4 * NW, 1, 0)

        def chunk_body(ci, carry):
            chunk = wid + ci * NW
            base = chunk * BLK
            pltpu.sync_copy(idxb_hbm.at[chunk], idx_v)
            pltpu.sync_copy(init_hbm.at[pl.ds(base, BLK)], acc_v)

            def fire(g, c):
                sub = lax.rem(g, SUB)
                src = h_hbm.at[idx_v.at[pl.ds(g * G, G)]]
                dst = acc_v.at[pl.ds(sub * G, G)]
                pltpu.async_copy(src, dst, sem, add=True)
                return c

            lax.fori_loop(0, K * SUB, fire, 0)

            def drain(g, c):
                pltpu.make_async_copy(
                    h_hbm.at[idx_v.at[pl.ds(0, G)]], acc_v.at[pl.ds(0, G)], sem
                ).wait()
                return c

            lax.fori_loop(0, K * SUB, drain, 0)
            pltpu.sync_copy(acc_v, out_hbm.at[pl.ds(base, BLK)])
            return carry

        lax.fori_loop(0, nch_w, chunk_body, 0)

    return conv(hflat, idxb, init)


def kernel(x, neighbor_idx, neighbor_mask, W1, b1, W2, b2,
           gamma1, beta1, gamma2, beta2):
    f32 = jnp.float32
    idx = neighbor_idx.astype(jnp.int32)
    offs = jnp.arange(K, dtype=jnp.int32)[None, :]
    nzpad = (NPAD - N) * K
    rowv = jnp.arange(N, dtype=jnp.int32)[:, None]
    sentinel = N * K + (rowv * K + offs) % nzpad
    idxc = jnp.where(neighbor_mask != 0, idx * K + offs, sentinel)
    idxc = jnp.pad(idxc, ((0, NPAD - N), (0, 0)), constant_values=N * K)
    idxb = idxc.reshape(NCH, BLK, K).transpose(0, 2, 1).reshape(NCH, K * BLK)

    xp = jnp.pad(x.astype(f32), ((0, NPAD - N), (0, 0)))
    zero_init = jnp.zeros((NPAD, C), f32)

    w1r = W1.astype(f32).transpose(1, 0, 2).reshape(C, K * C)
    w2r = W2.astype(f32).transpose(1, 0, 2).reshape(C, K * C)
    bvec1 = jnp.zeros((1, K * C), f32)
    bvec2 = jnp.zeros((K * C,), f32).at[KC * C:(KC + 1) * C].set(b2).reshape(1, K * C)

    st1 = _stats(xp)
    h1 = _transform(xp, st1, gamma1, beta1, w1r, bvec1)
    out1 = _sc_conv(h1.reshape(NPAD * K, C), idxb, zero_init)
    st2 = _stats(out1)
    h2 = _transform(out1, st2, gamma2, beta2, w2r, bvec2)
    out2 = _sc_conv(h2.reshape(NPAD * K, C), idxb, xp)
    return out2[:N]
